# Initial kernel scaffold; baseline (speedup 1.0000x reference)
import functools

import jax
import jax.numpy as jnp
from jax import lax
from jax.experimental import pallas as pl
from jax.experimental.pallas import tpu as pltpu
from jax.experimental.pallas import tpu_sc as plsc

NC = 2
NS = 16
NW = NC * NS
L = 16

CHUNK = 80
GROUPS = CHUNK // L


def _tec_body(D, per_w, table_hbm, src_hbm, tgt_hbm, out_hbm,
              sidx_v, tidx_v, srows_v, trows_v, acc_v, out_v, sem):
  wid = lax.axis_index("s") * NC + lax.axis_index("c")
  base = wid * per_w
  n_chunks = per_w // CHUNK
  nslice = D // L

  def chunk_body(ci, carry):
    cbase = base + ci * CHUNK
    pltpu.sync_copy(src_hbm.at[pl.ds(cbase, CHUNK)], sidx_v)
    pltpu.sync_copy(tgt_hbm.at[pl.ds(cbase, CHUNK)], tidx_v)
    pltpu.async_copy(table_hbm.at[sidx_v], srows_v, sem).wait()
    pltpu.async_copy(table_hbm.at[tidx_v], trows_v, sem).wait()

    def group_body(g, c2):
      eb = g * L
      for j in range(L):
        e = eb + j
        acc = srows_v[e, pl.ds(0, L)] * trows_v[e, pl.ds(0, L)]
        for k in range(1, nslice):
          acc = acc + srows_v[e, pl.ds(k * L, L)] * trows_v[e, pl.ds(k * L, L)]
        acc_v[j, :] = acc
      rows = lax.iota(jnp.int32, L)
      tot = plsc.load_gather(acc_v, [rows, jnp.zeros((L,), jnp.int32)])
      for i in range(1, L):
        tot = tot + plsc.load_gather(acc_v, [rows, jnp.full((L,), i, jnp.int32)])
      out_v[pl.ds(eb, L)] = 1.0 / (1.0 + jnp.exp(-tot))
      return c2

    lax.fori_loop(0, GROUPS, group_body, 0)
    pltpu.sync_copy(out_v, out_hbm.at[pl.ds(cbase, CHUNK)])
    return carry

  lax.fori_loop(0, n_chunks, chunk_body, 0)


def _link_predict(table, src, tgt):
  E = src.shape[0]
  D = table.shape[1]
  assert E % NW == 0
  per_w = E // NW
  assert per_w % CHUNK == 0 and D % L == 0

  mesh = plsc.VectorSubcoreMesh(core_axis_name="c", subcore_axis_name="s")
  k = pl.kernel(
      functools.partial(_tec_body, D, per_w),
      out_type=jax.ShapeDtypeStruct((E,), jnp.float32),
      mesh=mesh,
      scratch_types=[
          pltpu.VMEM((CHUNK,), jnp.int32),
          pltpu.VMEM((CHUNK,), jnp.int32),
          pltpu.VMEM((CHUNK, D), jnp.float32),
          pltpu.VMEM((CHUNK, D), jnp.float32),
          pltpu.VMEM((L, L), jnp.float32),
          pltpu.VMEM((CHUNK,), jnp.float32),
          pltpu.SemaphoreType.DMA,
      ],
  )
  return k(table, src, tgt)


def kernel(node_embedding_matrix, pos_edge_index, neg_edge_index, batch_train_x_index):
  src = jnp.concatenate([pos_edge_index[0], neg_edge_index[0]]).astype(jnp.int32)
  tgt = jnp.concatenate([pos_edge_index[1], neg_edge_index[1]]).astype(jnp.int32)
  return _link_predict(node_embedding_matrix, src, tgt)


# speedup vs baseline: 4.4783x; 4.4783x over previous
import functools

import jax
import jax.numpy as jnp
from jax import lax
from jax.experimental import pallas as pl
from jax.experimental.pallas import tpu as pltpu
from jax.experimental.pallas import tpu_sc as plsc

NC = 2
NS = 16
NW = NC * NS
L = 16

CHUNK = 80
GROUPS = CHUNK // L


def _tec_body(D, per_w, table_hbm, src_hbm, tgt_hbm, out_hbm,
              sidx_v, tidx_v, srows_v, trows_v, acc_v, out_v, sem):
  wid = lax.axis_index("s") * NC + lax.axis_index("c")
  base = wid * per_w
  n_chunks = per_w // CHUNK
  nslice = D // L

  def chunk_body(ci, carry):
    cbase = base + ci * CHUNK
    pltpu.sync_copy(src_hbm.at[pl.ds(cbase, CHUNK)], sidx_v)
    pltpu.sync_copy(tgt_hbm.at[pl.ds(cbase, CHUNK)], tidx_v)
    pltpu.async_copy(table_hbm.at[sidx_v], srows_v, sem).wait()
    pltpu.async_copy(table_hbm.at[tidx_v], trows_v, sem).wait()

    def group_body(g, c2):
      eb = g * L
      for j in range(L):
        e = eb + j
        acc = srows_v[e, pl.ds(0, L)] * trows_v[e, pl.ds(0, L)]
        for k in range(1, nslice):
          acc = acc + srows_v[e, pl.ds(k * L, L)] * trows_v[e, pl.ds(k * L, L)]
        acc_v[pl.ds(j * L, L)] = acc
      rows = lax.iota(jnp.int32, L) * L
      tot = plsc.load_gather(acc_v, [rows])
      for i in range(1, L):
        tot = tot + plsc.load_gather(acc_v, [rows + i])
      out_v[pl.ds(eb, L)] = 1.0 / (1.0 + jnp.exp(-tot))
      return c2

    lax.fori_loop(0, GROUPS, group_body, 0)
    pltpu.sync_copy(out_v, out_hbm.at[pl.ds(cbase, CHUNK)])
    return carry

  lax.fori_loop(0, n_chunks, chunk_body, 0)


def _link_predict(table, src, tgt):
  E = src.shape[0]
  D = table.shape[1]
  assert E % NW == 0
  per_w = E // NW
  assert per_w % CHUNK == 0 and D % L == 0

  mesh = plsc.VectorSubcoreMesh(core_axis_name="c", subcore_axis_name="s")
  k = pl.kernel(
      functools.partial(_tec_body, D, per_w),
      out_type=jax.ShapeDtypeStruct((E,), jnp.float32),
      mesh=mesh,
      compiler_params=pltpu.CompilerParams(needs_layout_passes=False),
      scratch_types=[
          pltpu.VMEM((CHUNK,), jnp.int32),
          pltpu.VMEM((CHUNK,), jnp.int32),
          pltpu.VMEM((CHUNK, D), jnp.float32),
          pltpu.VMEM((CHUNK, D), jnp.float32),
          pltpu.VMEM((L * L,), jnp.float32),
          pltpu.VMEM((CHUNK,), jnp.float32),
          pltpu.SemaphoreType.DMA,
      ],
  )
  return k(table, src, tgt)


def kernel(node_embedding_matrix, pos_edge_index, neg_edge_index, batch_train_x_index):
  src = jnp.concatenate([pos_edge_index[0], neg_edge_index[0]]).astype(jnp.int32)
  tgt = jnp.concatenate([pos_edge_index[1], neg_edge_index[1]]).astype(jnp.int32)
  return _link_predict(node_embedding_matrix, src, tgt)


# speedup vs baseline: 12.0559x; 2.6920x over previous
import functools

import jax
import jax.numpy as jnp
from jax import lax
from jax.experimental import pallas as pl
from jax.experimental.pallas import tpu as pltpu
from jax.experimental.pallas import tpu_sc as plsc

NC = 2
NS = 16
NW = NC * NS
L = 16

CHUNK = 80
GROUPS = CHUNK // L
NBUF = 2


def _tec_body(D, per_w, table_hbm, src_hbm, tgt_hbm, out_hbm,
              sidx_all, tidx_all, srows0, trows0, srows1, trows1,
              acc_v, out_v, sem0, sem1):
  wid = lax.axis_index("s") * NC + lax.axis_index("c")
  n_chunks = per_w // CHUNK
  base = wid * per_w
  nslice = D // L
  bufs = ((srows0, trows0, sem0), (srows1, trows1, sem1))

  pltpu.sync_copy(src_hbm.at[pl.ds(base, per_w)], sidx_all)
  pltpu.sync_copy(tgt_hbm.at[pl.ds(base, per_w)], tidx_all)

  def fire(ci, b):
    srows, trows, sem = bufs[b]
    off = pl.multiple_of(ci * CHUNK, 8)
    pltpu.async_copy(table_hbm.at[sidx_all.at[pl.ds(off, CHUNK)]], srows, sem)
    pltpu.async_copy(table_hbm.at[tidx_all.at[pl.ds(off, CHUNK)]], trows, sem)

  for b in range(NBUF):
    fire(b, b)

  def compute(ci, srows, trows):
    def group_body(g, c2):
      eb = g * L
      for j in range(L):
        e = eb + j
        acc = srows[e, pl.ds(0, L)] * trows[e, pl.ds(0, L)]
        for k in range(1, nslice):
          acc = acc + srows[e, pl.ds(k * L, L)] * trows[e, pl.ds(k * L, L)]
        acc_v[pl.ds(j * L, L)] = acc
      rows = lax.iota(jnp.int32, L) * L
      tot = plsc.load_gather(acc_v, [rows])
      for i in range(1, L):
        tot = tot + plsc.load_gather(acc_v, [rows + i])
      out_v[pl.ds(ci * CHUNK + eb, L)] = 1.0 / (1.0 + jnp.exp(-tot))
      return c2

    lax.fori_loop(0, GROUPS, group_body, 0)

  def outer(io, carry):
    for b in range(NBUF):
      ci = io * NBUF + b
      srows, trows, sem = bufs[b]
      off = pl.multiple_of(ci * CHUNK, 8)
      pltpu.make_async_copy(table_hbm.at[sidx_all.at[pl.ds(off, CHUNK)]], srows, sem).wait()
      pltpu.make_async_copy(table_hbm.at[tidx_all.at[pl.ds(off, CHUNK)]], trows, sem).wait()
      compute(ci, srows, trows)

      @pl.when(ci + NBUF < n_chunks)
      def _():
        fire(ci + NBUF, b)

    return carry

  lax.fori_loop(0, n_chunks // NBUF, outer, 0)
  pltpu.sync_copy(out_v, out_hbm.at[pl.ds(wid * per_w, per_w)])


def _link_predict(table, src, tgt):
  E = src.shape[0]
  D = table.shape[1]
  assert E % NW == 0
  per_w = E // NW
  n_chunks = per_w // CHUNK
  assert per_w % CHUNK == 0 and D % L == 0 and n_chunks % NBUF == 0

  mesh = plsc.VectorSubcoreMesh(core_axis_name="c", subcore_axis_name="s")
  k = pl.kernel(
      functools.partial(_tec_body, D, per_w),
      out_type=jax.ShapeDtypeStruct((E,), jnp.float32),
      mesh=mesh,
      compiler_params=pltpu.CompilerParams(needs_layout_passes=False),
      scratch_types=[
          pltpu.VMEM((per_w,), jnp.int32),
          pltpu.VMEM((per_w,), jnp.int32),
          pltpu.VMEM((CHUNK, D), jnp.float32),
          pltpu.VMEM((CHUNK, D), jnp.float32),
          pltpu.VMEM((CHUNK, D), jnp.float32),
          pltpu.VMEM((CHUNK, D), jnp.float32),
          pltpu.VMEM((L * L,), jnp.float32),
          pltpu.VMEM((per_w,), jnp.float32),
          pltpu.SemaphoreType.DMA,
          pltpu.SemaphoreType.DMA,
      ],
  )
  return k(table, src, tgt)


def kernel(node_embedding_matrix, pos_edge_index, neg_edge_index, batch_train_x_index):
  src = jnp.concatenate([pos_edge_index[0], neg_edge_index[0]]).astype(jnp.int32)
  tgt = jnp.concatenate([pos_edge_index[1], neg_edge_index[1]]).astype(jnp.int32)
  return _link_predict(node_embedding_matrix, src, tgt)
